# baseline (speedup 1.0000x reference)
import functools

import jax
import jax.numpy as jnp
from jax import lax
from jax.experimental import pallas as pl
from jax.experimental.pallas import tpu as pltpu
from jax.experimental.pallas import tpu_sc as plsc

_NCODES = 1024
_D = 64
_BLK = 4096

_SC_CORES = 2
_SC_SUBCORES = 16
_SC_WORKERS = _SC_CORES * _SC_SUBCORES
_GATHER_CHUNK = 128


def _vq_tc_body(z_ref, cb_ref, cin_ref, lin_ref,
                idx_ref, cout_ref, lout_ref, *scalar_refs,
                finalize, ntok_total):
    step = pl.program_id(0)
    nsteps = pl.num_programs(0)

    z = z_ref[...]
    cb = cb_ref[...]

    z2 = jnp.sum(z * z, axis=1, keepdims=True)
    c2 = jnp.sum(cb * cb, axis=1)
    m = lax.dot_general(z, cb, (((1,), (1,)), ((), ())),
                        preferred_element_type=jnp.float32)
    d = (z2 + c2[None, :]) - 2.0 * m

    dmin = jnp.min(d, axis=1, keepdims=True)
    minmask = d == dmin
    iota = lax.broadcasted_iota(jnp.int32, (_BLK, _NCODES), 1)
    idx = jnp.min(jnp.where(minmask, iota, _NCODES), axis=1,
                  keepdims=True)
    idx_ref[...] = idx.reshape(1, _BLK).reshape(1, 1, _BLK)

    part_counts = lax.dot_general(
        jnp.ones((1, _BLK), jnp.float32), minmask.astype(jnp.float32),
        (((1,), (0,)), ((), ())),
        preferred_element_type=jnp.float32)
    part_loss = jnp.sum(dmin)

    @pl.when(step == 0)
    def _init():
        cout_ref[...] = cin_ref[...] + part_counts
        lout_ref[...] = lin_ref[...] + part_loss

    @pl.when(step != 0)
    def _acc():
        cout_ref[...] += part_counts
        lout_ref[...] = lout_ref[...] + part_loss

    if finalize:
        loss_ref, perp_ref = scalar_refs

        @pl.when(step == nsteps - 1)
        def _fin():
            loss_ref[...] = lout_ref[...] / (ntok_total * _D)
            p = cout_ref[...] / ntok_total
            s = jnp.sum(p * jnp.log(p + 1e-10), axis=1, keepdims=True)
            perp_ref[...] = jnp.exp(-s)


def _vq_tc_half(z_half, codebook, counts_in, losssum_in, finalize,
                ntok_total):
    ntok = z_half.shape[0]
    grid = ntok // _BLK
    scalar_specs = []
    scalar_shapes = []
    if finalize:
        scalar_specs = [pl.BlockSpec((1, 1), lambda i: (0, 0)),
                        pl.BlockSpec((1, 1), lambda i: (0, 0))]
        scalar_shapes = [jax.ShapeDtypeStruct((1, 1), jnp.float32),
                         jax.ShapeDtypeStruct((1, 1), jnp.float32)]
    return pl.pallas_call(
        functools.partial(_vq_tc_body, finalize=finalize,
                          ntok_total=ntok_total),
        grid=(grid,),
        in_specs=[
            pl.BlockSpec((_BLK, _D), lambda i: (i, 0)),
            pl.BlockSpec((_NCODES, _D), lambda i: (0, 0)),
            pl.BlockSpec((1, _NCODES), lambda i: (0, 0)),
            pl.BlockSpec((1, 1), lambda i: (0, 0)),
        ],
        out_specs=[
            pl.BlockSpec((1, 1, _BLK), lambda i: (i, 0, 0)),
            pl.BlockSpec((1, _NCODES), lambda i: (0, 0)),
            pl.BlockSpec((1, 1), lambda i: (0, 0)),
        ] + scalar_specs,
        out_shape=[
            jax.ShapeDtypeStruct((grid, 1, _BLK), jnp.int32),
            jax.ShapeDtypeStruct((1, _NCODES), jnp.float32),
            jax.ShapeDtypeStruct((1, 1), jnp.float32),
        ] + scalar_shapes,
    )(z_half, codebook, counts_in, losssum_in)


def _sc_gather_body(cb_hbm, idx_hbm, out_hbm, idx_v, rows_v, sem):
    wid = lax.axis_index("s") * _SC_CORES + lax.axis_index("c")
    n_chunks = idx_hbm.shape[0] // _SC_WORKERS
    rows_per_worker = n_chunks * _GATHER_CHUNK
    base_chunk = wid * n_chunks
    pltpu.sync_copy(idx_hbm.at[pl.ds(base_chunk, n_chunks)], idx_v)
    copies = []
    for c in range(n_chunks):
        copies.append(pltpu.async_copy(
            cb_hbm.at[idx_v.at[c]],
            rows_v.at[pl.ds(c * _GATHER_CHUNK, _GATHER_CHUNK)], sem))
    for cp in copies:
        cp.wait()
    pltpu.sync_copy(rows_v,
                    out_hbm.at[pl.ds(wid * rows_per_worker, rows_per_worker)])


def _sc_gather(codebook, idx_chunks, ntok):
    rows_per_worker = ntok // _SC_WORKERS
    run = pl.kernel(
        _sc_gather_body,
        mesh=plsc.VectorSubcoreMesh(core_axis_name="c", subcore_axis_name="s"),
        out_type=jax.ShapeDtypeStruct((ntok, _D), jnp.float32),
        scratch_types=[
            pltpu.VMEM((ntok // _GATHER_CHUNK // _SC_WORKERS, _GATHER_CHUNK),
                       jnp.int32),
            pltpu.VMEM((rows_per_worker, _D), jnp.float32),
            pltpu.SemaphoreType.DMA,
        ],
        compiler_params=pltpu.CompilerParams(use_tc_tiling_on_sc=False),
    )
    return run(codebook, idx_chunks)


def kernel(z_e, codebook):
    shape = z_e.shape
    flat = z_e.reshape(-1, _D)
    ntok = flat.shape[0]
    half = ntok // 2

    zero_counts = jnp.zeros((1, _NCODES), jnp.float32)
    zero_loss = jnp.zeros((1, 1), jnp.float32)

    idx_a, counts_a, losssum_a = _vq_tc_half(
        flat[:half], codebook, zero_counts, zero_loss, False, ntok)
    zq_a = _sc_gather(codebook, idx_a.reshape(-1, _GATHER_CHUNK), half)

    idx_b, _, _, loss, perp = _vq_tc_half(
        flat[half:], codebook, counts_a, losssum_a, True, ntok)
    zq_b = _sc_gather(codebook, idx_b.reshape(-1, _GATHER_CHUNK), half)

    z_q_st = jnp.concatenate([zq_a, zq_b], axis=0).reshape(shape)
    indices_r = jnp.concatenate([idx_a, idx_b], axis=0).reshape(shape[:-1])
    loss_s = loss[0, 0]
    return (z_q_st, indices_r, loss_s, loss_s, perp[0, 0])


# speedup vs baseline: 1.1603x; 1.1603x over previous
import jax
import jax.numpy as jnp
from jax import lax
from jax.experimental import pallas as pl
from jax.experimental.pallas import tpu as pltpu
from jax.experimental.pallas import tpu_sc as plsc

_NCODES = 1024
_D = 64
_BLK = 4096

_SC_CORES = 2
_SC_SUBCORES = 16
_SC_WORKERS = _SC_CORES * _SC_SUBCORES
_GATHER_CHUNK = 128


def _vq_tc_body(z_ref, cb_ref, idx_ref, loss_ref, perp_ref,
                counts_ref, losssum_ref):
    step = pl.program_id(0)
    nsteps = pl.num_programs(0)
    ntok = nsteps * _BLK

    z = z_ref[...]
    cb = cb_ref[...]

    z2 = jnp.sum(z * z, axis=1, keepdims=True)
    c2 = jnp.sum(cb * cb, axis=1)
    m = lax.dot_general(z, cb, (((1,), (1,)), ((), ())),
                        preferred_element_type=jnp.float32)
    d = (z2 + c2[None, :]) - 2.0 * m

    dmin = jnp.min(d, axis=1, keepdims=True)
    minmask = d == dmin
    iota = lax.broadcasted_iota(jnp.int32, (_BLK, _NCODES), 1)
    idx = jnp.min(jnp.where(minmask, iota, _NCODES), axis=1,
                  keepdims=True)
    idx_ref[...] = idx.reshape(1, _BLK).reshape(1, 1, _BLK)

    part_counts = lax.dot_general(
        jnp.ones((1, _BLK), jnp.float32), minmask.astype(jnp.float32),
        (((1,), (0,)), ((), ())),
        preferred_element_type=jnp.float32)
    part_loss = jnp.sum(dmin)

    @pl.when(step == 0)
    def _init():
        counts_ref[...] = jnp.zeros_like(counts_ref)
        losssum_ref[...] = jnp.zeros_like(losssum_ref)

    counts_ref[...] += part_counts
    losssum_ref[...] = losssum_ref[...] + part_loss

    @pl.when(step == nsteps - 1)
    def _fin():
        loss_ref[...] = losssum_ref[...] / (ntok * _D)
        p = counts_ref[...] / ntok
        s = jnp.sum(p * jnp.log(p + 1e-10), axis=1, keepdims=True)
        perp_ref[...] = jnp.exp(-s)


def _sc_gather_body(cb_hbm, idx_hbm, out_hbm, idx_v, rows_v, sem):
    wid = lax.axis_index("s") * _SC_CORES + lax.axis_index("c")
    n_chunks = idx_hbm.shape[0] // _SC_WORKERS
    rows_per_worker = n_chunks * _GATHER_CHUNK
    base_chunk = wid * n_chunks
    pltpu.sync_copy(idx_hbm.at[pl.ds(base_chunk, n_chunks)], idx_v)
    copies = []
    for c in range(n_chunks):
        copies.append(pltpu.async_copy(
            cb_hbm.at[idx_v.at[c]],
            rows_v.at[pl.ds(c * _GATHER_CHUNK, _GATHER_CHUNK)], sem))
    for cp in copies:
        cp.wait()
    pltpu.sync_copy(rows_v,
                    out_hbm.at[pl.ds(wid * rows_per_worker, rows_per_worker)])


def _sc_gather(codebook, idx_chunks, ntok):
    rows_per_worker = ntok // _SC_WORKERS
    run = pl.kernel(
        _sc_gather_body,
        mesh=plsc.VectorSubcoreMesh(core_axis_name="c", subcore_axis_name="s"),
        out_type=jax.ShapeDtypeStruct((ntok, _D), jnp.float32),
        scratch_types=[
            pltpu.VMEM((ntok // _GATHER_CHUNK // _SC_WORKERS, _GATHER_CHUNK),
                       jnp.int32),
            pltpu.VMEM((rows_per_worker, _D), jnp.float32),
            pltpu.SemaphoreType.DMA,
        ],
        compiler_params=pltpu.CompilerParams(use_tc_tiling_on_sc=False),
    )
    return run(codebook, idx_chunks)


def kernel(z_e, codebook):
    shape = z_e.shape
    flat = z_e.reshape(-1, _D)
    ntok = flat.shape[0]
    grid = ntok // _BLK

    idx3d, loss, perp = pl.pallas_call(
        _vq_tc_body,
        grid=(grid,),
        in_specs=[
            pl.BlockSpec((_BLK, _D), lambda i: (i, 0)),
            pl.BlockSpec((_NCODES, _D), lambda i: (0, 0)),
        ],
        out_specs=[
            pl.BlockSpec((1, 1, _BLK), lambda i: (i, 0, 0)),
            pl.BlockSpec((1, 1), lambda i: (0, 0)),
            pl.BlockSpec((1, 1), lambda i: (0, 0)),
        ],
        out_shape=[
            jax.ShapeDtypeStruct((grid, 1, _BLK), jnp.int32),
            jax.ShapeDtypeStruct((1, 1), jnp.float32),
            jax.ShapeDtypeStruct((1, 1), jnp.float32),
        ],
        scratch_shapes=[
            pltpu.VMEM((1, _NCODES), jnp.float32),
            pltpu.VMEM((1, 1), jnp.float32),
        ],
    )(flat, codebook)

    zq = _sc_gather(codebook, idx3d.reshape(-1, _GATHER_CHUNK), ntok)

    z_q_st = zq.reshape(shape)
    indices_r = idx3d.reshape(shape[:-1])
    loss_s = loss[0, 0]
    return (z_q_st, indices_r, loss_s, loss_s, perp[0, 0])
